# baseline (speedup 1.0000x reference)
import functools

import jax
import jax.numpy as jnp
from jax import lax
from jax.experimental import pallas as pl
from jax.experimental.pallas import tpu as pltpu
from jax.experimental.pallas import tpu_sc as plsc

N = 200000
B = 160
NB = N // B
NW = 32
H = 20.0
GPB = B // 16

_SF_BLK = B * 64
_GR_BLK = B * 192


def _weights(t):
    s = 1.0 - t
    t2 = t * t
    t3 = t2 * t
    s2 = s * s
    s3 = s2 * s
    w0 = s3 * (1.0 / 6.0)
    w1 = 0.5 * t3 - t2 + 2.0 / 3.0
    w2 = 0.5 * s3 - s2 + 2.0 / 3.0
    w3 = t3 * (1.0 / 6.0)
    d0 = s2 * (-0.5 * H)
    d1 = (1.5 * H * t - 2.0 * H) * t
    d2 = (2.0 * H - 1.5 * H * s) * s
    d3 = t2 * (0.5 * H)
    return (w0, w1, w2, w3), (d0, d1, d2, d3)


def _frac(r):
    f = r - r.astype(jnp.int32).astype(jnp.float32)
    return jnp.where(f < 0.0, f + 1.0, f)


_mesh = plsc.VectorSubcoreMesh(core_axis_name="c", subcore_axis_name="s")


@functools.partial(
    pl.kernel,
    mesh=_mesh,
    out_type=[
        jax.ShapeDtypeStruct((N * 64,), jnp.float32),
        jax.ShapeDtypeStruct((N * 192,), jnp.float32),
    ],
    scratch_types=[
        pltpu.VMEM((3 * B,), jnp.float32),
        pltpu.VMEM((_SF_BLK,), jnp.float32),
        pltpu.VMEM((_GR_BLK,), jnp.float32),
    ],
    compiler_params=pltpu.CompilerParams(needs_layout_passes=False),
)
def _sc_shapef(xs, ys, zs, sf_out, gr_out, pv, sfb, grb):
    w = lax.axis_index("s") * 2 + lax.axis_index("c")
    nb = jnp.where(w < 2, 40, 39)
    start = w * 39 + jnp.minimum(w, 2)
    iota = lax.iota(jnp.int32, 16)
    i64 = iota * 64
    i192 = iota * 192

    def block_body(k, carry):
        row0 = (start + k) * B
        pltpu.sync_copy(xs.at[pl.ds(row0, B)], pv.at[pl.ds(0, B)])
        pltpu.sync_copy(ys.at[pl.ds(row0, B)], pv.at[pl.ds(B, B)])
        pltpu.sync_copy(zs.at[pl.ds(row0, B)], pv.at[pl.ds(2 * B, B)])

        def group_body(g, carry2):
            p0 = g * 16
            tx = _frac(pv[pl.ds(p0, 16)] * H)
            ty = _frac(pv[pl.ds(B + p0, 16)] * H)
            tz = _frac(pv[pl.ds(2 * B + p0, 16)] * H)
            wx, dwx = _weights(tx)
            wy, dwy = _weights(ty)
            wz, dwz = _weights(tz)
            base_sf = i64 + p0 * 64
            base_gr = i192 + p0 * 192
            for a0 in range(4):
                for a1 in range(4):
                    xy = wx[a0] * wy[a1]
                    dxy = dwx[a0] * wy[a1]
                    xdy = wx[a0] * dwy[a1]
                    for a2 in range(4):
                        j = a0 * 16 + a1 * 4 + a2
                        plsc.store_scatter(sfb, [base_sf + j], xy * wz[a2])
                        plsc.store_scatter(grb, [base_gr + 3 * j], dxy * wz[a2])
                        plsc.store_scatter(grb, [base_gr + (3 * j + 1)], xdy * wz[a2])
                        plsc.store_scatter(grb, [base_gr + (3 * j + 2)], xy * dwz[a2])
            return carry2

        lax.fori_loop(0, GPB, group_body, 0)
        pltpu.sync_copy(sfb, sf_out.at[pl.ds(row0 * 64, _SF_BLK)])
        pltpu.sync_copy(grb, gr_out.at[pl.ds(row0 * 192, _GR_BLK)])
        return carry

    lax.fori_loop(0, nb, block_body, 0)


def kernel(position_stack):
    pos = position_stack.astype(jnp.float32)
    sf_flat, gr_flat = _sc_shapef(pos[:, 0], pos[:, 1], pos[:, 2])
    return sf_flat.reshape(N, 64), gr_flat.reshape(N, 64, 3)


# speedup vs baseline: 42.0486x; 42.0486x over previous
import functools

import jax
import jax.numpy as jnp
from jax import lax
from jax.experimental import pallas as pl
from jax.experimental.pallas import tpu as pltpu
from jax.experimental.pallas import tpu_sc as plsc

N = 200000
W = 128
NFULL = N // W
TAIL = N - NFULL * W
H = 20.0


def _weights(t):
    s = 1.0 - t
    t2 = t * t
    t3 = t2 * t
    s2 = s * s
    s3 = s2 * s
    w0 = s3 * (1.0 / 6.0)
    w1 = 0.5 * t3 - t2 + 2.0 / 3.0
    w2 = 0.5 * s3 - s2 + 2.0 / 3.0
    w3 = t3 * (1.0 / 6.0)
    d0 = s2 * (-0.5 * H)
    d1 = (1.5 * H * t - 2.0 * H) * t
    d2 = (2.0 * H - 1.5 * H * s) * s
    d3 = t2 * (0.5 * H)
    return (w0, w1, w2, w3), (d0, d1, d2, d3)


def _frac(r):
    f = r - r.astype(jnp.int32).astype(jnp.float32)
    return jnp.where(f < 0.0, f + 1.0, f)


_mesh = plsc.VectorSubcoreMesh(core_axis_name="c", subcore_axis_name="s")


@functools.partial(
    pl.kernel,
    mesh=_mesh,
    out_type=[
        jax.ShapeDtypeStruct((64, N), jnp.float32),
        jax.ShapeDtypeStruct((3, 64, N), jnp.float32),
    ],
    scratch_types=[
        pltpu.VMEM((3 * W,), jnp.float32),
        pltpu.VMEM((64, W), jnp.float32),
        pltpu.VMEM((3, 64, W), jnp.float32),
    ],
    compiler_params=pltpu.CompilerParams(needs_layout_passes=False),
)
def _sc_shapef(xs, ys, zs, sf_out, gr_out, pv, sfb, grb):
    wid = lax.axis_index("s") * 2 + lax.axis_index("c")
    nb = jnp.where(wid < 26, 49, 48)

    def compute_block(c0, width):
        pltpu.sync_copy(xs.at[pl.ds(c0, width)], pv.at[pl.ds(0, width)])
        pltpu.sync_copy(ys.at[pl.ds(c0, width)], pv.at[pl.ds(W, width)])
        pltpu.sync_copy(zs.at[pl.ds(c0, width)], pv.at[pl.ds(2 * W, width)])

        def group_body(g, carry):
            p0 = g * 16
            tx = _frac(pv[pl.ds(p0, 16)] * H)
            ty = _frac(pv[pl.ds(W + p0, 16)] * H)
            tz = _frac(pv[pl.ds(2 * W + p0, 16)] * H)
            wx, dwx = _weights(tx)
            wy, dwy = _weights(ty)
            wz, dwz = _weights(tz)
            for a0 in range(4):
                for a1 in range(4):
                    xy = wx[a0] * wy[a1]
                    dxy = dwx[a0] * wy[a1]
                    xdy = wx[a0] * dwy[a1]
                    for a2 in range(4):
                        j = a0 * 16 + a1 * 4 + a2
                        sfb[j, pl.ds(p0, 16)] = xy * wz[a2]
                        grb[0, j, pl.ds(p0, 16)] = dxy * wz[a2]
                        grb[1, j, pl.ds(p0, 16)] = xdy * wz[a2]
                        grb[2, j, pl.ds(p0, 16)] = xy * dwz[a2]
            return carry

        lax.fori_loop(0, width // 16, group_body, 0)

    def block_body(k, carry):
        c0 = (wid + 32 * k) * W
        compute_block(c0, W)
        pltpu.sync_copy(sfb, sf_out.at[:, pl.ds(c0, W)])
        pltpu.sync_copy(grb, gr_out.at[:, :, pl.ds(c0, W)])
        return carry

    lax.fori_loop(0, nb, block_body, 0)

    @pl.when(wid == 31)
    def _tail():
        c0 = NFULL * W
        compute_block(c0, TAIL)
        def row_copy(j, carry):
            pltpu.sync_copy(sfb.at[j, pl.ds(0, TAIL)], sf_out.at[j, pl.ds(c0, TAIL)])
            pltpu.sync_copy(grb.at[0, j, pl.ds(0, TAIL)], gr_out.at[0, j, pl.ds(c0, TAIL)])
            pltpu.sync_copy(grb.at[1, j, pl.ds(0, TAIL)], gr_out.at[1, j, pl.ds(c0, TAIL)])
            pltpu.sync_copy(grb.at[2, j, pl.ds(0, TAIL)], gr_out.at[2, j, pl.ds(c0, TAIL)])
            return carry

        lax.fori_loop(0, 64, row_copy, 0)


def kernel(position_stack):
    pos = position_stack.astype(jnp.float32)
    sf_t, gr_t = _sc_shapef(pos[:, 0], pos[:, 1], pos[:, 2])
    return sf_t.T, gr_t.transpose(2, 1, 0)


# speedup vs baseline: 42.9096x; 1.0205x over previous
import functools

import jax
import jax.numpy as jnp
from jax import lax
from jax.experimental import pallas as pl
from jax.experimental.pallas import tpu as pltpu
from jax.experimental.pallas import tpu_sc as plsc

N = 200000
W = 128
NFULL = N // W
TAIL = N - NFULL * W
H = 20.0


def _weights(t):
    s = 1.0 - t
    t2 = t * t
    t3 = t2 * t
    s2 = s * s
    s3 = s2 * s
    w0 = s3 * (1.0 / 6.0)
    w1 = 0.5 * t3 - t2 + 2.0 / 3.0
    w2 = 0.5 * s3 - s2 + 2.0 / 3.0
    w3 = t3 * (1.0 / 6.0)
    d0 = s2 * (-0.5 * H)
    d1 = (1.5 * H * t - 2.0 * H) * t
    d2 = (2.0 * H - 1.5 * H * s) * s
    d3 = t2 * (0.5 * H)
    return (w0, w1, w2, w3), (d0, d1, d2, d3)


def _frac(r):
    f = r - r.astype(jnp.int32).astype(jnp.float32)
    return jnp.where(f < 0.0, f + 1.0, f)


_mesh = plsc.VectorSubcoreMesh(core_axis_name="c", subcore_axis_name="s")


@functools.partial(
    pl.kernel,
    mesh=_mesh,
    out_type=[
        jax.ShapeDtypeStruct((64, N), jnp.float32),
        jax.ShapeDtypeStruct((3, 64, N), jnp.float32),
    ],
    scratch_types=[
        pltpu.VMEM((2, 3 * W), jnp.float32),
        pltpu.VMEM((2, 64, W), jnp.float32),
        pltpu.VMEM((2, 3, 64, W), jnp.float32),
        pltpu.SemaphoreType.DMA((2,)),
        pltpu.SemaphoreType.DMA((2,)),
    ],
    compiler_params=pltpu.CompilerParams(needs_layout_passes=False),
)
def _sc_shapef(xs, ys, zs, sf_out, gr_out, pv, sfb, grb, in_sem, out_sem):
    wid = lax.axis_index("s") * 2 + lax.axis_index("c")
    nb = jnp.where(wid < 26, 49, 48)

    def in_copies(k, slot):
        c0 = (wid + 32 * k) * W
        return (
            pltpu.make_async_copy(xs.at[pl.ds(c0, W)], pv.at[slot, pl.ds(0, W)], in_sem.at[slot]),
            pltpu.make_async_copy(ys.at[pl.ds(c0, W)], pv.at[slot, pl.ds(W, W)], in_sem.at[slot]),
            pltpu.make_async_copy(zs.at[pl.ds(c0, W)], pv.at[slot, pl.ds(2 * W, W)], in_sem.at[slot]),
        )

    def out_copies(k, slot):
        c0 = (wid + 32 * k) * W
        return (
            pltpu.make_async_copy(sfb.at[slot], sf_out.at[:, pl.ds(c0, W)], out_sem.at[slot]),
            pltpu.make_async_copy(grb.at[slot], gr_out.at[:, :, pl.ds(c0, W)], out_sem.at[slot]),
        )

    def compute_groups(slot, ngroups):
        def group_body(g, carry):
            p0 = g * 16
            tx = _frac(pv[slot, pl.ds(p0, 16)] * H)
            ty = _frac(pv[slot, pl.ds(W + p0, 16)] * H)
            tz = _frac(pv[slot, pl.ds(2 * W + p0, 16)] * H)
            wx, dwx = _weights(tx)
            wy, dwy = _weights(ty)
            wz, dwz = _weights(tz)
            for a0 in range(4):
                for a1 in range(4):
                    xy = wx[a0] * wy[a1]
                    dxy = dwx[a0] * wy[a1]
                    xdy = wx[a0] * dwy[a1]
                    for a2 in range(4):
                        j = a0 * 16 + a1 * 4 + a2
                        sfb[slot, j, pl.ds(p0, 16)] = xy * wz[a2]
                        grb[slot, 0, j, pl.ds(p0, 16)] = dxy * wz[a2]
                        grb[slot, 1, j, pl.ds(p0, 16)] = xdy * wz[a2]
                        grb[slot, 2, j, pl.ds(p0, 16)] = xy * dwz[a2]
            return carry

        lax.fori_loop(0, ngroups, group_body, 0)

    for c in in_copies(0, 0):
        c.start()

    def block_body(k, carry):
        slot = lax.rem(k, 2)

        @pl.when(k + 1 < nb)
        def _prefetch():
            for c in in_copies(k + 1, 1 - slot):
                c.start()

        for c in in_copies(k, slot):
            c.wait()

        @pl.when(k >= 2)
        def _drain_out():
            for c in out_copies(k - 2, slot):
                c.wait()

        compute_groups(slot, W // 16)
        for c in out_copies(k, slot):
            c.start()
        return carry

    lax.fori_loop(0, nb, block_body, 0)

    for c in out_copies(nb - 2, lax.rem(nb - 2, 2)):
        c.wait()
    for c in out_copies(nb - 1, lax.rem(nb - 1, 2)):
        c.wait()

    @pl.when(wid == 31)
    def _tail():
        c0 = NFULL * W
        pltpu.sync_copy(xs.at[pl.ds(c0, TAIL)], pv.at[0, pl.ds(0, TAIL)])
        pltpu.sync_copy(ys.at[pl.ds(c0, TAIL)], pv.at[0, pl.ds(W, TAIL)])
        pltpu.sync_copy(zs.at[pl.ds(c0, TAIL)], pv.at[0, pl.ds(2 * W, TAIL)])
        compute_groups(0, TAIL // 16)
        def row_copy(j, carry):
            pltpu.sync_copy(sfb.at[0, j, pl.ds(0, TAIL)], sf_out.at[j, pl.ds(c0, TAIL)])
            pltpu.sync_copy(grb.at[0, 0, j, pl.ds(0, TAIL)], gr_out.at[0, j, pl.ds(c0, TAIL)])
            pltpu.sync_copy(grb.at[0, 1, j, pl.ds(0, TAIL)], gr_out.at[1, j, pl.ds(c0, TAIL)])
            pltpu.sync_copy(grb.at[0, 2, j, pl.ds(0, TAIL)], gr_out.at[2, j, pl.ds(c0, TAIL)])
            return carry

        lax.fori_loop(0, 64, row_copy, 0)


def kernel(position_stack):
    pos = position_stack.astype(jnp.float32)
    sf_t, gr_t = _sc_shapef(pos[:, 0], pos[:, 1], pos[:, 2])
    return sf_t.T, gr_t.transpose(2, 1, 0)
